# Initial kernel scaffold; baseline (speedup 1.0000x reference)
import jax
import jax.numpy as jnp
from jax.experimental import pallas as pl


def kernel(x, edge_index, edge_type, W0, al0, ar0, W1, al1, ar1, W2, al2, ar2, W3, al3, ar3):
    raise NotImplementedError("write your pallas kernel here")


# speedup vs baseline: 67.8004x; 67.8004x over previous
import functools

import numpy as np

import jax
import jax.numpy as jnp
from jax import lax
from jax.experimental import pallas as pl
from jax.experimental.pallas import tpu as pltpu
from jax.experimental.pallas import tpu_sc as plsc

N = 10000
E = 320000
D = 128
R = 8
H = 8
HD = 16

TN = 400
NT = N // TN
CH = 128
NCH = E // CH
NW = 32
KMAX = -(-NCH // NW)
NP = 10240
ROWS_PER_TILE = NP // 16

F32 = jnp.float32
HIGH = jax.lax.Precision.HIGHEST


def _lrelu(t):
    return jnp.maximum(t, 0.2 * t)


def _a_body(h_ref, w_ref, al_ref, ar_ref, z_ref, el_ref, er_ref, gelp_ref):
    zb = jnp.dot(h_ref[...], w_ref[0], precision=HIGH)
    z_ref[0] = zb
    elb = jnp.dot(zb, al_ref[...], precision=HIGH)
    el_ref[0] = elb
    er_ref[0] = jnp.dot(zb, ar_ref[...], precision=HIGH)
    gelp_ref[0, 0] = jnp.max(elb, axis=0, keepdims=True)


_kernel_a = pl.pallas_call(
    _a_body,
    grid=(NT, R),
    in_specs=[
        pl.BlockSpec((TN, D), lambda n, r: (n, 0)),
        pl.BlockSpec((1, D, D), lambda n, r: (r, 0, 0)),
        pl.BlockSpec((D, H), lambda n, r: (0, 0)),
        pl.BlockSpec((D, H), lambda n, r: (0, 0)),
    ],
    out_specs=[
        pl.BlockSpec((1, TN, D), lambda n, r: (r, n, 0)),
        pl.BlockSpec((1, TN, H), lambda n, r: (r, n, 0)),
        pl.BlockSpec((1, TN, H), lambda n, r: (r, n, 0)),
        pl.BlockSpec((1, 1, 1, H), lambda n, r: (n, r, 0, 0)),
    ],
    out_shape=[
        jax.ShapeDtypeStruct((R, N, D), F32),
        jax.ShapeDtypeStruct((R, N, H), F32),
        jax.ShapeDtypeStruct((R, N, H), F32),
        jax.ShapeDtypeStruct((NT, R, 1, H), F32),
    ],
)


def _a2_body(gelp_ref, er_ref, mh_ref):
    gel = jnp.max(gelp_ref[...], axis=(0, 2))
    er = er_ref[...]
    t = _lrelu(gel[:, None, :] + er)
    mh_ref[...] = jnp.max(t, axis=0)


_kernel_a2 = pl.pallas_call(
    _a2_body,
    grid=(NT,),
    in_specs=[
        pl.BlockSpec((NT, R, 1, H), lambda n: (0, 0, 0, 0)),
        pl.BlockSpec((R, TN, H), lambda n: (0, n, 0)),
    ],
    out_specs=pl.BlockSpec((TN, H), lambda n: (n, 0)),
    out_shape=jax.ShapeDtypeStruct((N, H), F32),
)


_sc_mesh = plsc.VectorSubcoreMesh(core_axis_name="c", subcore_axis_name="s")


@functools.partial(
    pl.kernel,
    out_type=(
        jax.ShapeDtypeStruct((2, NP * H), F32),
        jax.ShapeDtypeStruct((2, NP, D), F32),
    ),
    mesh=_sc_mesh,
    scratch_types=[
        pltpu.VMEM((CH,), jnp.int32),
        pltpu.VMEM((CH,), jnp.int32),
        pltpu.VMEM((CH,), jnp.int32),
        pltpu.VMEM((CH,), jnp.int32),
        pltpu.VMEM((H, CH), jnp.int32),
        pltpu.VMEM((H, CH), jnp.int32),
        pltpu.VMEM((H, CH), jnp.int32),
        pltpu.VMEM((H, CH), F32),
        pltpu.VMEM((H, CH), F32),
        pltpu.VMEM((H, CH), F32),
        pltpu.VMEM((H, CH), F32),
        pltpu.VMEM((CH, D), F32),
        pltpu.VMEM((1024,), F32),
        pltpu.VMEM_SHARED((NP * H,), F32),
        pltpu.VMEM_SHARED((NP, D), F32),
        pltpu.SemaphoreType.DMA,
    ],
)
def _edge_kernel(el_hbm, er_hbm, mh_hbm, z_hbm, src_hbm, dst_hbm, et_hbm,
                 s_out, hacc_out,
                 srcb, dstb, etb, idxrs, ixs, ixd, ixm, elg, erg, mg, wb, zg,
                 zb1, s_sp, hacc_sp, sem):
    cid = lax.axis_index("c")
    sid = lax.axis_index("s")
    wid = sid * 2 + cid
    zeros16 = jnp.zeros((16,), F32)

    def _zero_zg(i, _):
        for j in range(D // 16):
            zg[i, pl.ds(j * 16, 16)] = zeros16
        return 0
    lax.fori_loop(0, CH, _zero_zg, 0)

    def _zero_zb1(i, _):
        zb1[pl.ds(i * 16, 16)] = zeros16
        return 0
    lax.fori_loop(0, 64, _zero_zb1, 0)

    row0 = sid * ROWS_PER_TILE
    for t in range(ROWS_PER_TILE // CH):
        pltpu.sync_copy(zg, hacc_sp.at[pl.ds(row0 + t * CH, CH)])
        pltpu.sync_copy(zb1, s_sp.at[pl.ds((row0 + t * CH) * H, CH * H)])
    plsc.subcore_barrier()

    def _chunk(k, _):
        chunk_id = wid + NW * k

        @pl.when(chunk_id < NCH)
        def _body():
            base = chunk_id * CH
            pltpu.sync_copy(src_hbm.at[pl.ds(base, CH)], srcb)
            pltpu.sync_copy(dst_hbm.at[pl.ds(base, CH)], dstb)
            pltpu.sync_copy(et_hbm.at[pl.ds(base, CH)], etb)

            def _idx(i, _):
                sl = pl.ds(i * 16, 16)
                etN = etb[sl] * N
                rs = etN + srcb[sl]
                idxrs[sl] = rs
                rs8 = rs * H
                rd8 = (etN + dstb[sl]) * H
                dm8 = dstb[sl] * H
                for h in range(H):
                    ixs[h, sl] = rs8 + h
                    ixd[h, sl] = rd8 + h
                    ixm[h, sl] = dm8 + h
                return 0
            lax.fori_loop(0, CH // 16, _idx, 0)

            cz = pltpu.async_copy(z_hbm.at[idxrs], zg, sem)
            cps = [cz]
            for h in range(H):
                cps.append(pltpu.async_copy(el_hbm.at[ixs.at[h]], elg.at[h], sem))
                cps.append(pltpu.async_copy(er_hbm.at[ixd.at[h]], erg.at[h], sem))
                cps.append(pltpu.async_copy(mh_hbm.at[ixm.at[h]], mg.at[h], sem))
            for c in cps:
                c.wait()

            def _wloop(j, _):
                sl = pl.ds(j * 16, 16)
                for h in range(H):
                    t = elg[h, sl] + erg[h, sl]
                    wb[h, sl] = jnp.exp(_lrelu(t) - mg[h, sl])
                return 0
            lax.fori_loop(0, CH // 16, _wloop, 0)

            def _scale(cv, _):
                for h in range(H):
                    wv = wb[h, pl.ds(cv * 16, 16)]
                    for t in range(16):
                        whc = wv.at[jnp.full((16,), t, jnp.int32)].get(
                            mode='promise_in_bounds')
                        sl = pl.ds(h * 16, 16)
                        c = cv * 16 + t
                        zg[c, sl] = zg[c, sl] * whc
                return 0
            lax.fori_loop(0, CH // 16, _scale, 0)

            for h in range(H):
                pltpu.sync_copy(wb.at[h], s_sp.at[ixm.at[h]], add=True)
            pltpu.sync_copy(zg, hacc_sp.at[dstb], add=True)
        return 0
    lax.fori_loop(0, KMAX, _chunk, 0)

    plsc.subcore_barrier()
    pltpu.sync_copy(s_sp.at[pl.ds(row0 * H, ROWS_PER_TILE * H)],
                    s_out.at[cid].at[pl.ds(row0 * H, ROWS_PER_TILE * H)])
    pltpu.sync_copy(hacc_sp.at[pl.ds(row0, ROWS_PER_TILE)],
                    hacc_out.at[cid].at[pl.ds(row0, ROWS_PER_TILE)])


def _f_body(final, hacc_ref, s_ref, erep_ref, x_ref, out_ref):
    ha = hacc_ref[0] + hacc_ref[1]
    s8 = s_ref[0] + s_ref[1]
    sexp = jnp.dot(s8, erep_ref[...], precision=HIGH) + 1e-10
    h = _lrelu(ha / sexp)
    if final:
        h = h + x_ref[...]
    out_ref[...] = h


def _make_kernel_f(final):
    return pl.pallas_call(
        functools.partial(_f_body, final),
        grid=(NT,),
        in_specs=[
            pl.BlockSpec((2, TN, D), lambda n: (0, n, 0)),
            pl.BlockSpec((2, TN, H), lambda n: (0, n, 0)),
            pl.BlockSpec((H, D), lambda n: (0, 0)),
            pl.BlockSpec((TN, D), lambda n: (n, 0)),
        ],
        out_specs=pl.BlockSpec((TN, D), lambda n: (n, 0)),
        out_shape=jax.ShapeDtypeStruct((N, D), F32),
    )


_kernel_f_mid = _make_kernel_f(False)
_kernel_f_final = _make_kernel_f(True)


def _a_layout(a):
    idx = jnp.arange(D)
    head = idx // HD
    return jnp.zeros((D, H), F32).at[idx, head].set(a.reshape(-1))


def kernel(x, edge_index, edge_type, W0, al0, ar0, W1, al1, ar1,
           W2, al2, ar2, W3, al3, ar3):
    src = edge_index[0]
    dst = edge_index[1]
    et = edge_type
    params = [(W0, al0, ar0), (W1, al1, ar1), (W2, al2, ar2), (W3, al3, ar3)]

    erep = jnp.asarray(_EREP)
    h = x
    for l, (W, al, ar) in enumerate(params):
        z, el, er, gelp = _kernel_a(h, W, _a_layout(al), _a_layout(ar))
        mh = _kernel_a2(gelp, er)
        s2, hacc2 = _edge_kernel(el.reshape(-1), er.reshape(-1),
                                 mh.reshape(-1), z.reshape(R * N, D),
                                 src, dst, et)
        s2 = s2.reshape(2, NP, H)
        if l < 3:
            h = _kernel_f_mid(hacc2, s2, erep, x)
        else:
            h = _kernel_f_final(hacc2, s2, erep, x)
    return h


_EREP = np.repeat(np.eye(H, dtype=np.float32), HD, axis=1)


# speedup vs baseline: 90.8235x; 1.3396x over previous
import functools

import numpy as np

import jax
import jax.numpy as jnp
from jax import lax
from jax.experimental import pallas as pl
from jax.experimental.pallas import tpu as pltpu
from jax.experimental.pallas import tpu_sc as plsc

N = 10000
E = 320000
D = 128
R = 8
H = 8
HD = 16

TN = 400
NT = N // TN
CH = 64
NCH = E // CH
NW = 32
KMAX = -(-NCH // NW)
NP = 10240
ROWS_PER_TILE = NP // 16

F32 = jnp.float32
HIGH = jax.lax.Precision.HIGHEST


def _lrelu(t):
    return jnp.maximum(t, 0.2 * t)


def _a_body(h_ref, w_ref, al_ref, ar_ref, z_ref, el_ref, er_ref, gelp_ref):
    zb = jnp.dot(h_ref[...], w_ref[0], precision=HIGH)
    z_ref[0] = zb
    elb = jnp.dot(zb, al_ref[...], precision=HIGH)
    el_ref[0] = elb
    er_ref[0] = jnp.dot(zb, ar_ref[...], precision=HIGH)
    gelp_ref[0, 0] = jnp.max(elb, axis=0, keepdims=True)


_kernel_a = pl.pallas_call(
    _a_body,
    grid=(NT, R),
    in_specs=[
        pl.BlockSpec((TN, D), lambda n, r: (n, 0)),
        pl.BlockSpec((1, D, D), lambda n, r: (r, 0, 0)),
        pl.BlockSpec((D, H), lambda n, r: (0, 0)),
        pl.BlockSpec((D, H), lambda n, r: (0, 0)),
    ],
    out_specs=[
        pl.BlockSpec((1, TN, D), lambda n, r: (r, n, 0)),
        pl.BlockSpec((1, TN, H), lambda n, r: (r, n, 0)),
        pl.BlockSpec((1, TN, H), lambda n, r: (r, n, 0)),
        pl.BlockSpec((1, 1, 1, H), lambda n, r: (n, r, 0, 0)),
    ],
    out_shape=[
        jax.ShapeDtypeStruct((R, N, D), F32),
        jax.ShapeDtypeStruct((R, N, H), F32),
        jax.ShapeDtypeStruct((R, N, H), F32),
        jax.ShapeDtypeStruct((NT, R, 1, H), F32),
    ],
)


def _a2_body(gelp_ref, er_ref, mh_ref):
    gel = jnp.max(gelp_ref[...], axis=(0, 2))
    er = er_ref[...]
    t = _lrelu(gel[:, None, :] + er)
    mh_ref[...] = jnp.max(t, axis=0)


_kernel_a2 = pl.pallas_call(
    _a2_body,
    grid=(NT,),
    in_specs=[
        pl.BlockSpec((NT, R, 1, H), lambda n: (0, 0, 0, 0)),
        pl.BlockSpec((R, TN, H), lambda n: (0, n, 0)),
    ],
    out_specs=pl.BlockSpec((TN, H), lambda n: (n, 0)),
    out_shape=jax.ShapeDtypeStruct((N, H), F32),
)


_sc_mesh = plsc.VectorSubcoreMesh(core_axis_name="c", subcore_axis_name="s")


_NBUF = 2
_KU = -(-(KMAX + 2) // _NBUF)


def _sc_scratch():
    per_set = [
        pltpu.VMEM((CH,), jnp.int32),
        pltpu.VMEM((CH,), jnp.int32),
        pltpu.VMEM((CH,), jnp.int32),
        pltpu.VMEM((CH,), jnp.int32),
        pltpu.VMEM((H, CH), jnp.int32),
        pltpu.VMEM((H, CH), jnp.int32),
        pltpu.VMEM((H, CH), jnp.int32),
        pltpu.VMEM((H, CH), F32),
        pltpu.VMEM((H, CH), F32),
        pltpu.VMEM((H, CH), F32),
        pltpu.VMEM((H, CH), F32),
        pltpu.VMEM((CH, D), F32),
        pltpu.SemaphoreType.DMA,
        pltpu.SemaphoreType.DMA,
    ]
    return per_set * _NBUF + [
        pltpu.VMEM((CH * H,), F32),
        pltpu.VMEM_SHARED((NP * H,), F32),
        pltpu.VMEM_SHARED((NP, D), F32),
    ]


@functools.partial(
    pl.kernel,
    out_type=(
        jax.ShapeDtypeStruct((2, NP * H), F32),
        jax.ShapeDtypeStruct((2, NP, D), F32),
    ),
    mesh=_sc_mesh,
    scratch_types=_sc_scratch(),
)
def _edge_kernel(el_hbm, er_hbm, mh_hbm, z_hbm, src_hbm, dst_hbm, et_hbm,
                 s_out, hacc_out, *scr):
    sets = [scr[i * 14:(i + 1) * 14] for i in range(_NBUF)]
    zb1, s_sp, hacc_sp = scr[_NBUF * 14:]
    cid = lax.axis_index("c")
    sid = lax.axis_index("s")
    wid = sid * 2 + cid
    zeros16 = jnp.zeros((16,), F32)

    zg0 = sets[0][11]

    def _zero_zg(i, _):
        for j in range(D // 16):
            zg0[i, pl.ds(j * 16, 16)] = zeros16
        return 0
    lax.fori_loop(0, CH, _zero_zg, 0)

    def _zero_zb1(i, _):
        zb1[pl.ds(i * 16, 16)] = zeros16
        return 0
    lax.fori_loop(0, CH * H // 16, _zero_zb1, 0)

    row0 = sid * ROWS_PER_TILE
    for t in range(ROWS_PER_TILE // CH):
        pltpu.sync_copy(zg0, hacc_sp.at[pl.ds(row0 + t * CH, CH)])
        pltpu.sync_copy(zb1, s_sp.at[pl.ds((row0 + t * CH) * H, CH * H)])
    plsc.subcore_barrier()

    def fire_gathers(s, chunk_id):
        (srcb, dstb, etb, idxrs, ixs, ixd, ixm,
         elg, erg, mg, wb, zg, gsem, ssem) = sets[s]
        base = chunk_id * CH
        pltpu.sync_copy(src_hbm.at[pl.ds(base, CH)], srcb)
        pltpu.sync_copy(dst_hbm.at[pl.ds(base, CH)], dstb)
        pltpu.sync_copy(et_hbm.at[pl.ds(base, CH)], etb)

        def _idx(i, _):
            sl = pl.ds(i * 16, 16)
            etN = etb[sl] * N
            rs = etN + srcb[sl]
            idxrs[sl] = rs
            rs8 = rs * H
            rd8 = (etN + dstb[sl]) * H
            dm8 = dstb[sl] * H
            for h in range(H):
                ixs[h, sl] = rs8 + h
                ixd[h, sl] = rd8 + h
                ixm[h, sl] = dm8 + h
            return 0
        lax.fori_loop(0, CH // 16, _idx, 0)
        for src, dst in _gather_pairs(s):
            pltpu.async_copy(src, dst, gsem)

    def _gather_pairs(s):
        (srcb, dstb, etb, idxrs, ixs, ixd, ixm,
         elg, erg, mg, wb, zg, gsem, ssem) = sets[s]
        pairs = [(z_hbm.at[idxrs], zg)]
        for h in range(H):
            pairs.append((el_hbm.at[ixs.at[h]], elg.at[h]))
            pairs.append((er_hbm.at[ixd.at[h]], erg.at[h]))
            pairs.append((mh_hbm.at[ixm.at[h]], mg.at[h]))
        return pairs

    def drain_gathers(s):
        gsem = sets[s][12]
        for src, dst in _gather_pairs(s):
            pltpu.make_async_copy(src, dst, gsem).wait()

    def _scatter_pairs(s):
        (srcb, dstb, etb, idxrs, ixs, ixd, ixm,
         elg, erg, mg, wb, zg, gsem, ssem) = sets[s]
        pairs = [(wb.at[h], s_sp.at[ixm.at[h]]) for h in range(H)]
        pairs.append((zg, hacc_sp.at[dstb]))
        return pairs

    def fire_scatters(s):
        ssem = sets[s][13]
        for src, dst in _scatter_pairs(s):
            pltpu.async_copy(src, dst, ssem, add=True)

    def drain_scatters(s):
        ssem = sets[s][13]
        for src, dst in _scatter_pairs(s):
            pltpu.make_async_copy(src, dst, ssem).wait()

    def compute(s):
        (srcb, dstb, etb, idxrs, ixs, ixd, ixm,
         elg, erg, mg, wb, zg, gsem, ssem) = sets[s]

        def _wloop(j, _):
            sl = pl.ds(j * 16, 16)
            for h in range(H):
                t = elg[h, sl] + erg[h, sl]
                wb[h, sl] = jnp.exp(_lrelu(t) - mg[h, sl])
            return 0
        lax.fori_loop(0, CH // 16, _wloop, 0)

        def _scale(cv, _):
            for h in range(H):
                wv = wb[h, pl.ds(cv * 16, 16)]
                sl = pl.ds(h * 16, 16)
                for t in range(16):
                    whc = wv.at[jnp.full((16,), t, jnp.int32)].get(
                        mode='promise_in_bounds')
                    c = cv * 16 + t
                    zg[c, sl] = zg[c, sl] * whc
            return 0
        lax.fori_loop(0, CH // 16, _scale, 0)

    fire_gathers(0, wid)

    def _group(k3, _):
        for j in range(_NBUF):
            i = _NBUF * k3 + j
            c_i = wid + NW * i
            c_ip1 = c_i + NW
            c_prev = wid + NW * (i + 1 - _NBUF)

            @pl.when((i + 1 - _NBUF >= 0) & (c_prev < NCH))
            def _():
                drain_scatters((j + 1) % _NBUF)

            @pl.when(c_ip1 < NCH)
            def _():
                fire_gathers((j + 1) % _NBUF, c_ip1)

            @pl.when(c_i < NCH)
            def _():
                drain_gathers(j)
                compute(j)
                fire_scatters(j)
        return 0
    lax.fori_loop(0, _KU, _group, 0)

    plsc.subcore_barrier()
    pltpu.sync_copy(s_sp.at[pl.ds(row0 * H, ROWS_PER_TILE * H)],
                    s_out.at[cid].at[pl.ds(row0 * H, ROWS_PER_TILE * H)])
    pltpu.sync_copy(hacc_sp.at[pl.ds(row0, ROWS_PER_TILE)],
                    hacc_out.at[cid].at[pl.ds(row0, ROWS_PER_TILE)])


def _f_body(final, hacc_ref, s_ref, erep_ref, x_ref, out_ref):
    ha = hacc_ref[0] + hacc_ref[1]
    s8 = s_ref[0] + s_ref[1]
    sexp = jnp.dot(s8, erep_ref[...], precision=HIGH) + 1e-10
    h = _lrelu(ha / sexp)
    if final:
        h = h + x_ref[...]
    out_ref[...] = h


def _make_kernel_f(final):
    return pl.pallas_call(
        functools.partial(_f_body, final),
        grid=(NT,),
        in_specs=[
            pl.BlockSpec((2, TN, D), lambda n: (0, n, 0)),
            pl.BlockSpec((2, TN, H), lambda n: (0, n, 0)),
            pl.BlockSpec((H, D), lambda n: (0, 0)),
            pl.BlockSpec((TN, D), lambda n: (n, 0)),
        ],
        out_specs=pl.BlockSpec((TN, D), lambda n: (n, 0)),
        out_shape=jax.ShapeDtypeStruct((N, D), F32),
    )


_kernel_f_mid = _make_kernel_f(False)
_kernel_f_final = _make_kernel_f(True)


def _a_layout(a):
    idx = jnp.arange(D)
    head = idx // HD
    return jnp.zeros((D, H), F32).at[idx, head].set(a.reshape(-1))


def kernel(x, edge_index, edge_type, W0, al0, ar0, W1, al1, ar1,
           W2, al2, ar2, W3, al3, ar3):
    src = edge_index[0]
    dst = edge_index[1]
    et = edge_type
    params = [(W0, al0, ar0), (W1, al1, ar1), (W2, al2, ar2), (W3, al3, ar3)]

    erep = jnp.asarray(_EREP)
    h = x
    for l, (W, al, ar) in enumerate(params):
        z, el, er, gelp = _kernel_a(h, W, _a_layout(al), _a_layout(ar))
        mh = _kernel_a2(gelp, er)
        s2, hacc2 = _edge_kernel(el.reshape(-1), er.reshape(-1),
                                 mh.reshape(-1), z.reshape(R * N, D),
                                 src, dst, et)
        s2 = s2.reshape(2, NP, H)
        if l < 3:
            h = _kernel_f_mid(hacc2, s2, erep, x)
        else:
            h = _kernel_f_final(hacc2, s2, erep, x)
    return h


_EREP = np.repeat(np.eye(H, dtype=np.float32), HD, axis=1)
